# baseline (speedup 1.0000x reference)
import functools

import jax
import jax.numpy as jnp
from jax import lax
from jax.experimental import pallas as pl
from jax.experimental.pallas import tpu as pltpu
from jax.experimental.pallas import tpu_sc as plsc

N = 10000
D = 128
E = 320000

NC, NS, L = 2, 16, 16
NW = NC * NS
NPAD = 10240
EROWS = 2560
ROWS_PER_W = EROWS // NW
PH = 40


def _deg_body(dst2d_hbm, ones_hbm, zeros_hbm, out_hbm, didx, ones_v, acc):
    c = lax.axis_index("c")
    s = lax.axis_index("s")
    wid = s * NC + c

    pltpu.sync_copy(ones_hbm, ones_v)

    @pl.when(s == 0)
    def _init():
        pltpu.sync_copy(zeros_hbm, acc)
    plsc.subcore_barrier()

    wbase = wid * ROWS_PER_W

    def chunk(k, _):
        pltpu.sync_copy(dst2d_hbm.at[pl.ds(wbase + k * 8, 8)], didx)
        for j in range(8):
            pltpu.sync_copy(ones_v, acc.at[didx.at[j]], add=True)
        return 0
    lax.fori_loop(0, ROWS_PER_W // 8, chunk, 0)

    plsc.subcore_barrier()

    @pl.when(s == 0)
    def _copyout():
        pltpu.sync_copy(acc, out_hbm.at[c])


def _edge_body(y_hbm, src2d_hbm, dst2d_hbm, zeros_hbm, out_hbm,
               sidx, didx, rows0, rows1, acc, sem0, sem1, ssem0, ssem1):
    c = lax.axis_index("c")
    s = lax.axis_index("s")
    wid = s * NC + c

    @pl.when(s == 0)
    def _init():
        pltpu.sync_copy(zeros_hbm, acc)
    plsc.subcore_barrier()

    wbase = wid * ROWS_PER_W

    for ph in range(ROWS_PER_W // PH):
        base = wbase + ph * PH
        pltpu.sync_copy(src2d_hbm.at[pl.ds(base, PH)], sidx)
        pltpu.sync_copy(dst2d_hbm.at[pl.ds(base, PH)], didx)
        pltpu.async_copy(y_hbm.at[sidx.at[0]], rows0, sem0)

        def body(t, _):
            j0 = 2 * t
            g1 = pltpu.async_copy(y_hbm.at[sidx.at[j0 + 1]], rows1, sem1)
            pltpu.make_async_copy(y_hbm.at[sidx.at[j0]], rows0, sem0).wait()
            s0 = pltpu.async_copy(rows0, acc.at[didx.at[j0]], ssem0, add=True)
            g1.wait()
            s1 = pltpu.async_copy(rows1, acc.at[didx.at[j0 + 1]], ssem1,
                                  add=True)
            s0.wait()

            @pl.when(j0 + 2 < PH)
            def _prefetch():
                pltpu.async_copy(y_hbm.at[sidx.at[j0 + 2]], rows0, sem0)

            s1.wait()
            return 0
        lax.fori_loop(0, PH // 2, body, 0)

    plsc.subcore_barrier()

    @pl.when(s == 0)
    def _copyout():
        pltpu.sync_copy(acc, out_hbm.at[c])


@functools.cache
def _sc_kernels():
    mesh = plsc.VectorSubcoreMesh(core_axis_name="c", subcore_axis_name="s",
                                  num_cores=NC, num_subcores=NS)
    deg = pl.kernel(
        _deg_body,
        out_type=jax.ShapeDtypeStruct((NC, NPAD, D), jnp.float32),
        mesh=mesh,
        scratch_types=[
            pltpu.VMEM((8, 128), jnp.int32),
            pltpu.VMEM((128, D), jnp.float32),
            pltpu.VMEM_SHARED((NPAD, D), jnp.float32),
        ],
    )
    edge = pl.kernel(
        _edge_body,
        out_type=jax.ShapeDtypeStruct((NC, NPAD, D), jnp.float32),
        mesh=mesh,
        scratch_types=[
            pltpu.VMEM((PH, 128), jnp.int32),
            pltpu.VMEM((PH, 128), jnp.int32),
            pltpu.VMEM((128, D), jnp.float32),
            pltpu.VMEM((128, D), jnp.float32),
            pltpu.VMEM_SHARED((NPAD, D), jnp.float32),
            pltpu.SemaphoreType.DMA,
            pltpu.SemaphoreType.DMA,
            pltpu.SemaphoreType.DMA,
            pltpu.SemaphoreType.DMA,
        ],
    )
    return deg, edge


def _tc1_body(x_ref, w1_ref, degp_ref, y1_ref, dinv_ref):
    cnt = degp_ref[0, :N, 0:1] + degp_ref[1, :N, 0:1]
    dinv = lax.rsqrt(cnt + 1.0)
    xw = jnp.dot(x_ref[...], w1_ref[...], preferred_element_type=jnp.float32)
    y1_ref[...] = xw * dinv
    dinv_ref[...] = dinv


def _tc2_body(p_ref, y1_ref, dinv_ref, b1_ref, w2_ref, y2_ref):
    dinv = dinv_ref[...]
    z = p_ref[0, :N, :] + p_ref[1, :N, :] + y1_ref[...]
    h = jnp.maximum(z * dinv + b1_ref[...][None, :], 0.0)
    y2_ref[...] = jnp.dot(h, w2_ref[...],
                          preferred_element_type=jnp.float32) * dinv


def _tc3_body(q_ref, y2_ref, dinv_ref, b2_ref, out_ref):
    z = q_ref[0, :N, :] + q_ref[1, :N, :] + y2_ref[...]
    out_ref[...] = z * dinv_ref[...] + b2_ref[...][None, :]


_tc1 = pl.pallas_call(
    _tc1_body,
    out_shape=(jax.ShapeDtypeStruct((N, D), jnp.float32),
               jax.ShapeDtypeStruct((N, 1), jnp.float32)),
)

_tc2 = pl.pallas_call(
    _tc2_body,
    out_shape=jax.ShapeDtypeStruct((N, D), jnp.float32),
)

_tc3 = pl.pallas_call(
    _tc3_body,
    out_shape=jax.ShapeDtypeStruct((N, D), jnp.float32),
)


def kernel(x, edge_index, W1, b1, W2, b2):
    ei = edge_index.astype(jnp.int32)
    npad_e = EROWS * 128 - E
    pad = jnp.arange(npad_e, dtype=jnp.int32)
    src2d = jnp.concatenate([ei[0], pad % N]).reshape(EROWS, 128)
    dst2d = jnp.concatenate(
        [ei[1], N + pad % (NPAD - N)]).reshape(EROWS, 128)

    deg_kernel, edge_kernel = _sc_kernels()
    zeros = jnp.zeros((NPAD, D), jnp.float32)
    ones = jnp.ones((128, D), jnp.float32)
    degp = deg_kernel(dst2d, ones, zeros)
    y1, dinv = _tc1(x, W1, degp)
    p = edge_kernel(y1, src2d, dst2d, zeros)
    y2 = _tc2(p, y1, dinv, b1, W2)
    q = edge_kernel(y2, src2d, dst2d, zeros)
    return _tc3(q, y2, dinv, b2)


# speedup vs baseline: 1.2035x; 1.2035x over previous
import functools

import jax
import jax.numpy as jnp
from jax import lax
from jax.experimental import pallas as pl
from jax.experimental.pallas import tpu as pltpu
from jax.experimental.pallas import tpu_sc as plsc

N = 10000
D = 128
E = 320000

NC, NS, L = 2, 16, 16
NW = NC * NS
NPAD = 10240
EROWS = 2560
ROWS_PER_W = EROWS // NW
PH = 40


def _deg_body(dst2d_hbm, ones_hbm, zeros_hbm, out_hbm, didx, ones_v, acc, dsem):
    c = lax.axis_index("c")
    s = lax.axis_index("s")
    wid = s * NC + c

    pltpu.sync_copy(ones_hbm, ones_v)

    @pl.when(s == 0)
    def _init():
        pltpu.sync_copy(zeros_hbm, acc)
    plsc.subcore_barrier()

    wbase = wid * ROWS_PER_W

    def chunk(k, _):
        pltpu.sync_copy(dst2d_hbm.at[pl.ds(wbase + k * 8, 8)], didx)
        cps = [pltpu.async_copy(ones_v, acc.at[didx.at[j]], dsem, add=True)
               for j in range(8)]
        for cp in cps:
            cp.wait()
        return 0
    lax.fori_loop(0, ROWS_PER_W // 8, chunk, 0)

    plsc.subcore_barrier()

    @pl.when(s == 0)
    def _copyout():
        pltpu.sync_copy(acc, out_hbm.at[c])


def _edge_body(y_hbm, src2d_hbm, dst2d_hbm, zeros_hbm, out_hbm,
               sidx, didx, rows0, rows1, acc, sem0, sem1):
    c = lax.axis_index("c")
    s = lax.axis_index("s")
    wid = s * NC + c

    @pl.when(s == 0)
    def _init():
        pltpu.sync_copy(zeros_hbm, acc)
    plsc.subcore_barrier()

    wbase = wid * ROWS_PER_W

    for ph in range(ROWS_PER_W // PH):
        base = wbase + ph * PH
        pltpu.sync_copy(src2d_hbm.at[pl.ds(base, PH)], sidx)
        pltpu.sync_copy(dst2d_hbm.at[pl.ds(base, PH)], didx)
        pltpu.async_copy(y_hbm.at[sidx.at[0]], rows0, sem0)

        def body(t, _):
            j0 = 2 * t
            cp1 = pltpu.async_copy(y_hbm.at[sidx.at[j0 + 1]], rows1, sem1)
            pltpu.make_async_copy(y_hbm.at[sidx.at[j0]], rows0, sem0).wait()
            pltpu.sync_copy(rows0, acc.at[didx.at[j0]], add=True)

            @pl.when(j0 + 2 < PH)
            def _prefetch():
                pltpu.async_copy(y_hbm.at[sidx.at[j0 + 2]], rows0, sem0)

            cp1.wait()
            pltpu.sync_copy(rows1, acc.at[didx.at[j0 + 1]], add=True)
            return 0
        lax.fori_loop(0, PH // 2, body, 0)

    plsc.subcore_barrier()

    @pl.when(s == 0)
    def _copyout():
        pltpu.sync_copy(acc, out_hbm.at[c])


@functools.cache
def _sc_kernels():
    mesh = plsc.VectorSubcoreMesh(core_axis_name="c", subcore_axis_name="s",
                                  num_cores=NC, num_subcores=NS)
    deg = pl.kernel(
        _deg_body,
        out_type=jax.ShapeDtypeStruct((NC, NPAD, D), jnp.float32),
        mesh=mesh,
        scratch_types=[
            pltpu.VMEM((8, 128), jnp.int32),
            pltpu.VMEM((128, D), jnp.float32),
            pltpu.VMEM_SHARED((NPAD, D), jnp.float32),
            pltpu.SemaphoreType.DMA,
        ],
    )
    edge = pl.kernel(
        _edge_body,
        out_type=jax.ShapeDtypeStruct((NC, NPAD, D), jnp.float32),
        mesh=mesh,
        scratch_types=[
            pltpu.VMEM((PH, 128), jnp.int32),
            pltpu.VMEM((PH, 128), jnp.int32),
            pltpu.VMEM((128, D), jnp.float32),
            pltpu.VMEM((128, D), jnp.float32),
            pltpu.VMEM_SHARED((NPAD, D), jnp.float32),
            pltpu.SemaphoreType.DMA,
            pltpu.SemaphoreType.DMA,
        ],
    )
    return deg, edge


def _tc1_body(x_ref, w1_ref, degp_ref, y1_ref, dinv_ref):
    cnt = degp_ref[0, :N, 0:1] + degp_ref[1, :N, 0:1]
    dinv = lax.rsqrt(cnt + 1.0)
    xw = jnp.dot(x_ref[...], w1_ref[...], preferred_element_type=jnp.float32)
    y1_ref[...] = xw * dinv
    dinv_ref[...] = dinv


def _tc2_body(p_ref, y1_ref, dinv_ref, b1_ref, w2_ref, y2_ref):
    dinv = dinv_ref[...]
    z = p_ref[0, :N, :] + p_ref[1, :N, :] + y1_ref[...]
    h = jnp.maximum(z * dinv + b1_ref[...][None, :], 0.0)
    y2_ref[...] = jnp.dot(h, w2_ref[...],
                          preferred_element_type=jnp.float32) * dinv


def _tc3_body(q_ref, y2_ref, dinv_ref, b2_ref, out_ref):
    z = q_ref[0, :N, :] + q_ref[1, :N, :] + y2_ref[...]
    out_ref[...] = z * dinv_ref[...] + b2_ref[...][None, :]


_tc1 = pl.pallas_call(
    _tc1_body,
    out_shape=(jax.ShapeDtypeStruct((N, D), jnp.float32),
               jax.ShapeDtypeStruct((N, 1), jnp.float32)),
)

_tc2 = pl.pallas_call(
    _tc2_body,
    out_shape=jax.ShapeDtypeStruct((N, D), jnp.float32),
)

_tc3 = pl.pallas_call(
    _tc3_body,
    out_shape=jax.ShapeDtypeStruct((N, D), jnp.float32),
)


def kernel(x, edge_index, W1, b1, W2, b2):
    ei = edge_index.astype(jnp.int32)
    npad_e = EROWS * 128 - E
    pad = jnp.arange(npad_e, dtype=jnp.int32)
    src2d = jnp.concatenate([ei[0], pad % N]).reshape(EROWS, 128)
    dst2d = jnp.concatenate(
        [ei[1], N + pad % (NPAD - N)]).reshape(EROWS, 128)

    deg_kernel, edge_kernel = _sc_kernels()
    zeros = jnp.zeros((NPAD, D), jnp.float32)
    ones = jnp.ones((128, D), jnp.float32)
    degp = deg_kernel(dst2d, ones, zeros)
    y1, dinv = _tc1(x, W1, degp)
    p = edge_kernel(y1, src2d, dst2d, zeros)
    y2 = _tc2(p, y1, dinv, b1, W2)
    q = edge_kernel(y2, src2d, dst2d, zeros)
    return _tc3(q, y2, dinv, b2)
